# baseline (speedup 1.0000x reference)
import functools

import jax
import jax.numpy as jnp
from jax.experimental import pallas as pl
from jax.experimental.pallas import tpu as pltpu

_MiB = 1024 * 1024


def _se_fused_kernel(w1e_ref, w2e_ref, b0_ref, al_ref, b1_ref, x_ref, o_ref,
                     *, inv_hw):
    xb = x_ref[0]

    p = jnp.sum(xb, axis=1, keepdims=True) * inv_hw

    y = jnp.sum(w1e_ref[...] * p, axis=0, keepdims=True)

    t = y + b0_ref[...]
    y = jnp.where(t >= 0.0, t, al_ref[...] * t) + b1_ref[...]

    y = jnp.sum(w2e_ref[...] * y, axis=1, keepdims=True)
    s = jax.nn.sigmoid(y).astype(o_ref.dtype)

    o_ref[0] = xb * s


def kernel(x, w1, w2, bias0, alpha, bias1):
    B, C, H, W = x.shape
    hw = H * W
    mid = w1.shape[0]

    x3 = x.reshape(B, C, hw)
    w1t = jnp.transpose(w1).astype(jnp.float32)
    w1e = jnp.mean(jnp.abs(w1t), axis=0, keepdims=True) * jnp.sign(w1t)
    w2f = w2.astype(jnp.float32)
    w2e = jnp.mean(jnp.abs(w2f), axis=1, keepdims=True) * jnp.sign(w2f)
    b0 = bias0.reshape(1, mid).astype(jnp.float32)
    al = alpha.reshape(1, mid).astype(jnp.float32)
    b1 = bias1.reshape(1, mid).astype(jnp.float32)

    itemsize = jnp.dtype(x.dtype).itemsize
    block_bytes = C * hw * itemsize
    vmem_limit = int(4 * block_bytes + 2 * C * mid * 4 + 8 * _MiB)

    vmem_full = pl.BlockSpec(memory_space=pltpu.MemorySpace.VMEM)
    fn = functools.partial(_se_fused_kernel, inv_hw=1.0 / float(hw))
    out3d = pl.pallas_call(
        fn,
        out_shape=jax.ShapeDtypeStruct((B, C, hw), x.dtype),
        grid_spec=pltpu.PrefetchScalarGridSpec(
            num_scalar_prefetch=0,
            grid=(B,),
            in_specs=[
                vmem_full, vmem_full,
                vmem_full, vmem_full, vmem_full,
                pl.BlockSpec((1, C, hw), lambda b: (b, 0, 0)),
            ],
            out_specs=pl.BlockSpec((1, C, hw), lambda b: (b, 0, 0)),
        ),
        compiler_params=pltpu.CompilerParams(
            dimension_semantics=("parallel",),
            vmem_limit_bytes=vmem_limit,
        ),
    )(w1e, w2e, b0, al, b1, x3)
    return out3d.reshape(B, C, H, W)


# speedup vs baseline: 1.1824x; 1.1824x over previous
import functools

import jax
import jax.numpy as jnp
from jax.experimental import pallas as pl
from jax.experimental.pallas import tpu as pltpu

_MiB = 1024 * 1024


def _se_fused_kernel(w1t_ref, w2_ref, b0_ref, al_ref, b1_ref, x_ref, o_ref,
                     *, inv_hw):
    xb = x_ref[...]

    p = jnp.sum(xb, axis=2) * inv_hw

    w1t = w1t_ref[...]
    sc1 = jnp.mean(jnp.abs(w1t), axis=0, keepdims=True)
    y = jnp.dot(p, sc1 * jnp.sign(w1t),
                preferred_element_type=jnp.float32)

    t = y + b0_ref[...]
    y = jnp.where(t >= 0.0, t, al_ref[...] * t) + b1_ref[...]

    w2 = w2_ref[...]
    sc2 = jnp.mean(jnp.abs(w2), axis=1, keepdims=True)
    s = jax.lax.dot_general(y, sc2 * jnp.sign(w2),
                            (((1,), (1,)), ((), ())),
                            preferred_element_type=jnp.float32)
    s = jax.nn.sigmoid(s).astype(o_ref.dtype)

    o_ref[...] = xb * s[:, :, None]


def kernel(x, w1, w2, bias0, alpha, bias1):
    B, C, H, W = x.shape
    hw = H * W
    mid = w1.shape[0]

    bb = 4 if B % 4 == 0 else (2 if B % 2 == 0 else 1)

    x3 = x.reshape(B, C, hw)
    w1t = jnp.transpose(w1).astype(jnp.float32)
    w2f = w2.astype(jnp.float32)
    b0 = bias0.reshape(1, mid).astype(jnp.float32)
    al = alpha.reshape(1, mid).astype(jnp.float32)
    b1 = bias1.reshape(1, mid).astype(jnp.float32)

    itemsize = jnp.dtype(x.dtype).itemsize
    block_bytes = bb * C * hw * itemsize
    vmem_limit = int(4 * block_bytes + 2 * C * mid * 4 + 8 * _MiB)

    vmem_full = pl.BlockSpec(memory_space=pltpu.MemorySpace.VMEM)
    fn = functools.partial(_se_fused_kernel, inv_hw=1.0 / float(hw))
    out3d = pl.pallas_call(
        fn,
        out_shape=jax.ShapeDtypeStruct((B, C, hw), x.dtype),
        grid_spec=pltpu.PrefetchScalarGridSpec(
            num_scalar_prefetch=0,
            grid=(B // bb,),
            in_specs=[
                vmem_full, vmem_full,
                vmem_full, vmem_full, vmem_full,
                pl.BlockSpec((bb, C, hw), lambda b: (b, 0, 0)),
            ],
            out_specs=pl.BlockSpec((bb, C, hw), lambda b: (b, 0, 0)),
        ),
        compiler_params=pltpu.CompilerParams(
            dimension_semantics=("parallel",),
            vmem_limit_bytes=vmem_limit,
        ),
    )(w1t, w2f, b0, al, b1, x3)
    return out3d.reshape(B, C, H, W)


# speedup vs baseline: 1.2139x; 1.0266x over previous
import functools

import jax
import jax.numpy as jnp
from jax.experimental import pallas as pl
from jax.experimental.pallas import tpu as pltpu

_MiB = 1024 * 1024


def _se_fused_kernel(w1t_ref, w2_ref, b0_ref, al_ref, b1_ref, x_ref, o_ref,
                     *, inv_hw):
    xb = x_ref[...]

    p = jnp.sum(xb, axis=2) * inv_hw

    w1t = w1t_ref[...]
    sc1 = jnp.mean(jnp.abs(w1t), axis=0, keepdims=True)
    y = jnp.dot(p, sc1 * jnp.sign(w1t),
                preferred_element_type=jnp.float32)

    t = y + b0_ref[...]
    y = jnp.where(t >= 0.0, t, al_ref[...] * t) + b1_ref[...]

    w2 = w2_ref[...]
    sc2 = jnp.mean(jnp.abs(w2), axis=1, keepdims=True)
    s = jax.lax.dot_general(y, sc2 * jnp.sign(w2),
                            (((1,), (1,)), ((), ())),
                            preferred_element_type=jnp.float32)
    s = jax.nn.sigmoid(s).astype(o_ref.dtype)

    o_ref[...] = xb * s[:, :, None]


def kernel(x, w1, w2, bias0, alpha, bias1):
    B, C, H, W = x.shape
    hw = H * W
    mid = w1.shape[0]

    bb = 8 if B % 8 == 0 else (4 if B % 4 == 0 else (2 if B % 2 == 0 else 1))

    x3 = x.reshape(B, C, hw)
    w1t = jnp.transpose(w1).astype(jnp.float32)
    w2f = w2.astype(jnp.float32)
    b0 = bias0.reshape(1, mid).astype(jnp.float32)
    al = alpha.reshape(1, mid).astype(jnp.float32)
    b1 = bias1.reshape(1, mid).astype(jnp.float32)

    itemsize = jnp.dtype(x.dtype).itemsize
    block_bytes = bb * C * hw * itemsize
    vmem_limit = int(4 * block_bytes + 2 * C * mid * 4 + 8 * _MiB)

    vmem_full = pl.BlockSpec(memory_space=pltpu.MemorySpace.VMEM)
    fn = functools.partial(_se_fused_kernel, inv_hw=1.0 / float(hw))
    out3d = pl.pallas_call(
        fn,
        out_shape=jax.ShapeDtypeStruct((B, C, hw), x.dtype),
        grid_spec=pltpu.PrefetchScalarGridSpec(
            num_scalar_prefetch=0,
            grid=(B // bb,),
            in_specs=[
                vmem_full, vmem_full,
                vmem_full, vmem_full, vmem_full,
                pl.BlockSpec((bb, C, hw), lambda b: (b, 0, 0)),
            ],
            out_specs=pl.BlockSpec((bb, C, hw), lambda b: (b, 0, 0)),
        ),
        compiler_params=pltpu.CompilerParams(
            dimension_semantics=("parallel",),
            vmem_limit_bytes=vmem_limit,
        ),
    )(w1t, w2f, b0, al, b1, x3)
    return out3d.reshape(B, C, H, W)
